# baseline (speedup 1.0000x reference)
import jax
import jax.numpy as jnp
from jax.experimental import pallas as pl
from jax.experimental.pallas import tpu as pltpu

_BM1 = 2048
_BM2 = 2048


def _pool_kernel(x_ref, w1, b1, w2, b2, w3, b3, w4, b4, out_ref):
    f32 = jnp.float32
    h = jnp.maximum(jnp.dot(x_ref[...], w1[...], preferred_element_type=f32) + b1[...], 0.0)
    h = jnp.maximum(jnp.dot(h, w2[...], preferred_element_type=f32) + b2[...], 0.0)
    y = jnp.maximum(jnp.dot(h, w3[...], preferred_element_type=f32) + b3[...], 0.0)
    y = jnp.dot(y, w4[...], preferred_element_type=f32)
    out_ref[0] = jnp.max(y, axis=0, keepdims=True)


def _head_kernel(x_ref, part_ref, w1, b1, w2, b2, b4, w5a, w5b, b5, w6, b6,
                 w7, b7, w8, b8, out_ref):
    f32 = jnp.float32
    g = jnp.maximum(jnp.max(part_ref[...], axis=0, keepdims=True) + b4[...], 0.0)
    c5 = jnp.dot(g, w5b[...], preferred_element_type=f32) + b5[...]
    h = jnp.maximum(jnp.dot(x_ref[...], w1[...], preferred_element_type=f32) + b1[...], 0.0)
    h = jnp.maximum(jnp.dot(h, w2[...], preferred_element_type=f32) + b2[...], 0.0)
    z = jnp.maximum(jnp.dot(h, w5a[...], preferred_element_type=f32) + c5, 0.0)
    z = jnp.maximum(jnp.dot(z, w6[...], preferred_element_type=f32) + b6[...], 0.0)
    z = jnp.maximum(jnp.dot(z, w7[...], preferred_element_type=f32) + b7[...], 0.0)
    out_ref[...] = jnp.dot(z, w8[...], preferred_element_type=f32) + b8[...]


def _full(shape):
    return pl.BlockSpec(shape, lambda i: tuple(0 for _ in shape))


def kernel(x, W1, b1, W2, b2, W3, b3, W4, b4, W5, b5, W6, b6, W7, b7, W8, b8):
    n = x.shape[2]
    pts = x.reshape(n, 3)
    w1t, w2t, w3t, w4t = W1.T, W2.T, W3.T, W4.T
    w5t = W5.T
    w5a, w5b = w5t[:64], w5t[64:]
    w6t, w7t, w8t = W6.T, W7.T, W8.T
    b1r, b2r, b3r, b4r = (b.reshape(1, -1) for b in (b1, b2, b3, b4))
    b5r, b6r, b7r, b8r = (b.reshape(1, -1) for b in (b5, b6, b7, b8))

    g1 = n // _BM1
    partials = pl.pallas_call(
        _pool_kernel,
        grid=(g1,),
        in_specs=[
            pl.BlockSpec((_BM1, 3), lambda i: (i, 0)),
            _full((3, 64)), _full((1, 64)),
            _full((64, 64)), _full((1, 64)),
            _full((64, 128)), _full((1, 128)),
            _full((128, 1024)), _full((1, 1024)),
        ],
        out_specs=pl.BlockSpec((1, 1, 1024), lambda i: (i, 0, 0)),
        out_shape=jax.ShapeDtypeStruct((g1, 1, 1024), jnp.float32),
        compiler_params=pltpu.CompilerParams(
            dimension_semantics=("parallel",),
        ),
        name="pointnet_pool",
    )(pts, w1t, b1r, w2t, b2r, w3t, b3r, w4t, b4r)

    part2 = partials.reshape(g1, 1024)

    g2 = n // _BM2
    out = pl.pallas_call(
        _head_kernel,
        grid=(g2,),
        in_specs=[
            pl.BlockSpec((_BM2, 3), lambda i: (i, 0)),
            _full((g1, 1024)),
            _full((3, 64)), _full((1, 64)),
            _full((64, 64)), _full((1, 64)),
            _full((1, 1024)),
            _full((64, 512)), _full((1024, 512)), _full((1, 512)),
            _full((512, 256)), _full((1, 256)),
            _full((256, 128)), _full((1, 128)),
            _full((128, 1)), _full((1, 1)),
        ],
        out_specs=pl.BlockSpec((_BM2, 1), lambda i: (i, 0)),
        out_shape=jax.ShapeDtypeStruct((n, 1), jnp.float32),
        compiler_params=pltpu.CompilerParams(
            dimension_semantics=("parallel",),
        ),
        name="pointnet_head",
    )(pts, part2, w1t, b1r, w2t, b2r, b4r, w5a, w5b, b5r, w6t, b6r, w7t, b7r,
      w8t, b8r)

    return out.reshape(1, 1, n, 1)


# speedup vs baseline: 1.1509x; 1.1509x over previous
import jax
import jax.numpy as jnp
from jax.experimental import pallas as pl
from jax.experimental.pallas import tpu as pltpu

_BM = 2048


def _dott(a, w):
    return jax.lax.dot_general(a, w, (((1,), (1,)), ((), ())),
                               preferred_element_type=jnp.float32)


def _kernel(x_ref, w1, b1, w2, b2, w3, b3, w4, b4, w5r, b5, w6, b6,
            w7, b7, w8, b8, out_ref, acc_ref, c5_ref):
    ph = pl.program_id(0)
    j = pl.program_id(1)

    @pl.when(ph == 0)
    def _pool():
        h = jnp.maximum(_dott(x_ref[...], w1[...]) + b1[...], 0.0)
        h = jnp.maximum(_dott(h, w2[...]) + b2[...], 0.0)
        y = jnp.maximum(_dott(h, w3[...]) + b3[...], 0.0)
        y = _dott(y, w4[...])
        bm = jnp.max(y, axis=0, keepdims=True)

        @pl.when(j == 0)
        def _():
            acc_ref[...] = bm

        @pl.when(j > 0)
        def _():
            acc_ref[...] = jnp.maximum(acc_ref[...], bm)

    @pl.when((ph == 1) & (j == 0))
    def _globals():
        g = jnp.maximum(acc_ref[...] + b4[...], 0.0)
        c5_ref[...] = _dott(g, w5r[:, :1024]) + b5[...]

    @pl.when(ph == 1)
    def _head():
        h = jnp.maximum(_dott(x_ref[...], w1[...]) + b1[...], 0.0)
        h = jnp.maximum(_dott(h, w2[...]) + b2[...], 0.0)
        z = jnp.maximum(_dott(h, w5r[:, 1024:]) + c5_ref[...], 0.0)
        z = jnp.maximum(_dott(z, w6[...]) + b6[...], 0.0)
        z = jnp.maximum(_dott(z, w7[...]) + b7[...], 0.0)
        out_ref[...] = jnp.sum(z * w8[...], axis=1, keepdims=True) + b8[...]


def _full(shape):
    return pl.BlockSpec(shape, lambda i, j: tuple(0 for _ in shape))


def kernel(x, W1, b1, W2, b2, W3, b3, W4, b4, W5, b5, W6, b6, W7, b7, W8, b8):
    n = x.shape[2]
    pts = x.reshape(n, 3)
    w5r = jnp.concatenate([W5[:, 64:], W5[:, :64]], axis=1)
    b1, b2, b3, b4 = (b.reshape(1, -1) for b in (b1, b2, b3, b4))
    b5, b6, b7, b8 = (b.reshape(1, -1) for b in (b5, b6, b7, b8))

    g2 = n // _BM
    out = pl.pallas_call(
        _kernel,
        grid=(2, g2),
        in_specs=[
            pl.BlockSpec((_BM, 3), lambda i, j: (j, 0)),
            _full((64, 3)), _full((1, 64)),
            _full((64, 64)), _full((1, 64)),
            _full((128, 64)), _full((1, 128)),
            _full((1024, 128)), _full((1, 1024)),
            _full((512, 1088)), _full((1, 512)),
            _full((256, 512)), _full((1, 256)),
            _full((128, 256)), _full((1, 128)),
            _full((1, 128)), _full((1, 1)),
        ],
        out_specs=pl.BlockSpec((_BM, 1), lambda i, j: (i * j, 0)),
        out_shape=jax.ShapeDtypeStruct((n, 1), jnp.float32),
        scratch_shapes=[
            pltpu.VMEM((1, 1024), jnp.float32),
            pltpu.VMEM((1, 512), jnp.float32),
        ],
        compiler_params=pltpu.CompilerParams(
            dimension_semantics=("arbitrary", "arbitrary"),
        ),
        name="pointnet_fused",
    )(pts, W1, b1, W2, b2, W3, b3, W4, b4, w5r, b5, W6, b6, W7, b7, W8, b8)

    return out.reshape(1, 1, n, 1)
